# baseline (speedup 1.0000x reference)
import functools

import numpy as np
import jax
import jax.numpy as jnp
from jax import lax
from jax.experimental import pallas as pl
from jax.experimental.pallas import tpu as pltpu
from jax.experimental.pallas import tpu_sc as plsc

_F32 = jnp.float32
_BN = 512
_LN = 16
_NC, _NS = 2, 16


def _tc_first(xT, Wt, Ms, Md):
    HC = Wt.shape[0]
    H = Ms.shape[0]
    NP = xT.shape[1]

    def body(x_ref, w_ref, ms_ref, md_ref, h_ref, a_ref):
        act = x_ref[...]
        h = jnp.dot(w_ref[...], act, preferred_element_type=_F32)
        a_s = jnp.dot(ms_ref[...], h, preferred_element_type=_F32)
        a_d = jnp.dot(md_ref[...], h, preferred_element_type=_F32)
        h_ref[...] = jnp.concatenate([h, jnp.ones((1, h.shape[1]), _F32)], axis=0)
        a_ref[...] = jnp.concatenate([a_s, a_d], axis=0)

    grid = (NP // _BN,)
    full = lambda a: pl.BlockSpec(a.shape, lambda j: (0, 0))
    col = lambda r: pl.BlockSpec((r, _BN), lambda j: (0, j))
    return pl.pallas_call(
        body,
        grid=grid,
        in_specs=[col(xT.shape[0]), full(Wt), full(Ms), full(Md)],
        out_specs=[col(HC + 1), col(2 * H)],
        out_shape=[
            jax.ShapeDtypeStruct((HC + 1, NP), _F32),
            jax.ShapeDtypeStruct((2 * H, NP), _F32),
        ],
    )(xT, Wt, Ms, Md)


def _tc_mid(sc_out, self_f, self_d, b_col, Wt, Ms, Md):
    HC = Wt.shape[0]
    H = Ms.shape[0]
    NP = sc_out.shape[1]

    def body(sc_ref, sf_ref, sd_ref, b_ref, w_ref, ms_ref, md_ref, h_ref, a_ref):
        sc = sc_ref[...]
        acc = jnp.dot(sf_ref[...], sc, preferred_element_type=_F32)
        den = jnp.dot(sd_ref[...], sc, preferred_element_type=_F32)
        act = acc / (den + 1e-16) + b_ref[...]
        act = jnp.where(act > 0, act, jnp.exp(act) - 1.0)
        h = jnp.dot(w_ref[...], act, preferred_element_type=_F32)
        a_s = jnp.dot(ms_ref[...], h, preferred_element_type=_F32)
        a_d = jnp.dot(md_ref[...], h, preferred_element_type=_F32)
        h_ref[...] = jnp.concatenate([h, jnp.ones((1, h.shape[1]), _F32)], axis=0)
        a_ref[...] = jnp.concatenate([a_s, a_d], axis=0)

    grid = (NP // _BN,)
    full = lambda a: pl.BlockSpec(a.shape, lambda j: (0, 0))
    col = lambda r: pl.BlockSpec((r, _BN), lambda j: (0, j))
    return pl.pallas_call(
        body,
        grid=grid,
        in_specs=[col(sc_out.shape[0]), full(self_f), full(self_d), full(b_col),
                  full(Wt), full(Ms), full(Md)],
        out_specs=[col(HC + 1), col(2 * H)],
        out_shape=[
            jax.ShapeDtypeStruct((HC + 1, NP), _F32),
            jax.ShapeDtypeStruct((2 * H, NP), _F32),
        ],
    )(sc_out, self_f, self_d, b_col, Wt, Ms, Md)


def _tc_final(sc_out, self_f, self_d, b_col):
    NP = sc_out.shape[1]
    CLS = self_f.shape[0]

    def body(sc_ref, sf_ref, sd_ref, b_ref, o_ref):
        sc = sc_ref[...]
        acc = jnp.dot(sf_ref[...], sc, preferred_element_type=_F32)
        den = jnp.dot(sd_ref[...], sc, preferred_element_type=_F32)
        val = acc / (den + 1e-16) + b_ref[...]
        m = jnp.max(val, axis=0, keepdims=True)
        ls = m + jnp.log(jnp.sum(jnp.exp(val - m), axis=0, keepdims=True))
        o_ref[...] = val - ls

    grid = (NP // _BN,)
    full = lambda a: pl.BlockSpec(a.shape, lambda j: (0, 0))
    col = lambda r: pl.BlockSpec((r, _BN), lambda j: (0, j))
    return pl.pallas_call(
        body,
        grid=grid,
        in_specs=[col(sc_out.shape[0]), full(self_f), full(self_d), full(b_col)],
        out_specs=col(CLS),
        out_shape=jax.ShapeDtypeStruct((CLS, NP), _F32),
    )(sc_out, self_f, self_d, b_col)


def _sc_layer(H, C, TPH, T, KMAX, NP, EP):
    VF = H * (C + 1)
    HC = H * C
    S = H * TPH
    EPT = EP // T
    CH = 2048 if T == 1 else 1024
    NCHUNK = EPT // CH
    assert EPT % CH == 0 and NCHUNK % 2 == 0 and S * T <= _NC * _NS
    NPAIR = NCHUNK // 2
    NGRP = CH // _LN

    mesh = plsc.VectorSubcoreMesh(core_axis_name="c", subcore_axis_name="s",
                                  num_cores=_NC, num_subcores=_NS)

    @functools.partial(
        pl.kernel,
        out_type=jax.ShapeDtypeStruct((T * VF, NP), _F32),
        mesh=mesh,
        compiler_params=pltpu.CompilerParams(needs_layout_passes=False,
                                             use_tc_tiling_on_sc=False),
        scratch_types=[
            pltpu.VMEM((NP,), _F32),
            pltpu.VMEM((NP,), _F32),
            pltpu.VMEM((KMAX, NP), _F32),
            pltpu.VMEM((KMAX, NP), _F32),
            pltpu.VMEM((4, CH), jnp.int32),
            pltpu.SemaphoreType.DMA((4,)),
        ],
    )
    def k(hT, aT, srcp, dstp, out, asrc_v, adst_v, hrow_v, acc_v, ebuf, sems):
        cid = lax.axis_index("c")
        sid = lax.axis_index("s")
        wid = sid * _NC + cid
        active = wid < S * T
        t = jnp.minimum(wid // S, T - 1)
        r = wid % S
        h = jnp.minimum(r // TPH, H - 1)
        slot = r % TPH
        e_base = t * EPT

        pltpu.sync_copy(aT.at[h], asrc_v)
        pltpu.sync_copy(aT.at[H + h], adst_v)
        for kk in range(KMAX):
            vf = slot + kk * TPH
            fr = jnp.where(vf < C, h * C + vf, HC)
            pltpu.sync_copy(hT.at[fr], hrow_v.at[kk])

        def zbody(i, _):
            z = jnp.zeros((_LN,), _F32)
            for kk in range(KMAX):
                acc_v[kk, pl.ds(i * _LN, _LN)] = z
            return 0
        lax.fori_loop(0, NP // _LN, zbody, 0)

        def start(buf, c):
            off = e_base + c * CH
            pltpu.make_async_copy(srcp.at[pl.ds(off, CH)], ebuf.at[buf],
                                  sems.at[buf]).start()
            pltpu.make_async_copy(dstp.at[pl.ds(off, CH)], ebuf.at[2 + buf],
                                  sems.at[2 + buf]).start()

        def wait(buf):
            pltpu.make_async_copy(srcp.at[pl.ds(0, CH)], ebuf.at[buf],
                                  sems.at[buf]).wait()
            pltpu.make_async_copy(dstp.at[pl.ds(0, CH)], ebuf.at[2 + buf],
                                  sems.at[2 + buf]).wait()

        def process(buf):
            @plsc.parallel_loop(0, NGRP, 1, unroll=4)
            def g_body(g):
                off = g * _LN
                si = ebuf[buf, pl.ds(off, _LN)]
                di = ebuf[2 + buf, pl.ds(off, _LN)]
                av = plsc.load_gather(asrc_v, [si]) + plsc.load_gather(adst_v, [di])
                av = jnp.where(av > 0, av, 0.2 * av)
                s = jnp.exp(av)
                for kk in range(KMAX):
                    rk = jnp.full((_LN,), kk, jnp.int32)
                    gv = plsc.load_gather(hrow_v, [rk, si])
                    plsc.addupdate_scatter(acc_v, [rk, di], gv * s)

        start(0, 0)

        def pair_body(p, _):
            c0 = p * 2
            wait(0)
            start(1, c0 + 1)
            process(0)
            wait(1)

            @pl.when(p < NPAIR - 1)
            def _():
                start(0, c0 + 2)
            process(1)
            return 0
        lax.fori_loop(0, NPAIR, pair_body, 0)

        for kk in range(KMAX):
            vf = slot + kk * TPH
            row = t * VF + h * (C + 1) + jnp.minimum(vf, C)

            @pl.when(active & (vf <= C))
            def _():
                pltpu.sync_copy(acc_v.at[kk], out.at[row])

    return k


def _sel_mats(H, C, T):
    VF = H * (C + 1)
    HC = H * C
    sf = np.zeros((HC, T * VF), np.float32)
    sd = np.zeros((HC, T * VF), np.float32)
    for hh in range(H):
        for cc in range(C):
            f = hh * C + cc
            for tt in range(T):
                sf[f, tt * VF + hh * (C + 1) + cc] = 1.0
                sd[f, tt * VF + hh * (C + 1) + C] = 1.0
    return jnp.asarray(sf), jnp.asarray(sd)


def _head_sum(H, C):
    p = np.zeros((H, H * C), np.float32)
    for hh in range(H):
        p[hh, hh * C:(hh + 1) * C] = 1.0
    return jnp.asarray(p)


_CFGS = [
    (10, 10, 3, 1, 4),
    (8, 10, 4, 1, 3),
    (4, 10, 4, 2, 3),
    (1, 10, 4, 8, 3),
    (1, 10, 4, 8, 3),
]


def kernel(x, edge_index, W1, a1s, a1d, b1, W2, a2s, a2d, b2,
           W3, a3s, a3d, b3, W4, a4s, a4d, b4, W5, a5s, a5d, b5):
    N, Fin = x.shape
    E0 = edge_index.shape[1]
    NP = ((N + _BN) // _BN) * _BN
    E_tot = E0 + N
    EP = ((E_tot + 16383) // 16384) * 16384

    idx = edge_index.astype(jnp.int32)
    ar = jnp.arange(N, dtype=jnp.int32)
    pad_e = jnp.full((EP - E_tot,), N, jnp.int32)
    srcp = jnp.concatenate([idx[0], ar, pad_e])
    dstp = jnp.concatenate([idx[1], ar, pad_e])

    xT = jnp.zeros((Fin, NP), _F32).at[:, :N].set(x.T)

    Ws = [W1, W2, W3, W4, W5]
    As = [a1s, a2s, a3s, a4s, a5s]
    Ad = [a1d, a2d, a3d, a4d, a5d]
    Bs = [b1, b2, b3, b4, b5]

    sc_out = None
    for i, (H, C, TPH, T, KMAX) in enumerate(_CFGS):
        Wt = Ws[i].T
        P = _head_sum(H, C)
        Ms = P * As[i].reshape(1, -1)
        Md = P * Ad[i].reshape(1, -1)
        if i == 0:
            hT, aT = _tc_first(xT, Wt, Ms, Md)
        else:
            Hp, Cp, _, Tp, _ = _CFGS[i - 1]
            sf, sd = _sel_mats(Hp, Cp, Tp)
            b_col = Bs[i - 1].reshape(-1, 1)
            hT, aT = _tc_mid(sc_out, sf, sd, b_col, Wt, Ms, Md)
        sc_out = _sc_layer(H, C, TPH, T, KMAX, NP, EP)(hT, aT, srcp, dstp)

    H, C, _, T, _ = _CFGS[-1]
    sf, sd = _sel_mats(H, C, T)
    logitsT = _tc_final(sc_out, sf, sd, b5.reshape(-1, 1))
    return logitsT.T[:N]


# speedup vs baseline: 1.0889x; 1.0889x over previous
import functools

import numpy as np
import jax
import jax.numpy as jnp
from jax import lax
from jax.experimental import pallas as pl
from jax.experimental.pallas import tpu as pltpu
from jax.experimental.pallas import tpu_sc as plsc

_F32 = jnp.float32
_BN = 512
_LN = 16
_NC, _NS = 2, 16


def _tc_first(xT, Wt, Ms, Md):
    HC = Wt.shape[0]
    H = Ms.shape[0]
    NP = xT.shape[1]

    def body(x_ref, w_ref, ms_ref, md_ref, h_ref, a_ref):
        act = x_ref[...]
        h = jnp.dot(w_ref[...], act, preferred_element_type=_F32)
        a_s = jnp.dot(ms_ref[...], h, preferred_element_type=_F32)
        a_d = jnp.dot(md_ref[...], h, preferred_element_type=_F32)
        h_ref[...] = jnp.concatenate([h, jnp.ones((1, h.shape[1]), _F32)], axis=0)
        a_ref[...] = jnp.concatenate([a_s, a_d], axis=0)

    grid = (NP // _BN,)
    full = lambda a: pl.BlockSpec(a.shape, lambda j: (0, 0))
    col = lambda r: pl.BlockSpec((r, _BN), lambda j: (0, j))
    return pl.pallas_call(
        body,
        grid=grid,
        in_specs=[col(xT.shape[0]), full(Wt), full(Ms), full(Md)],
        out_specs=[col(HC + 1), col(2 * H)],
        out_shape=[
            jax.ShapeDtypeStruct((HC + 1, NP), _F32),
            jax.ShapeDtypeStruct((2 * H, NP), _F32),
        ],
    )(xT, Wt, Ms, Md)


def _tc_mid(sc_out, self_f, self_d, b_col, Wt, Ms, Md):
    HC = Wt.shape[0]
    H = Ms.shape[0]
    NP = sc_out.shape[1]

    def body(sc_ref, sf_ref, sd_ref, b_ref, w_ref, ms_ref, md_ref, h_ref, a_ref):
        sc = sc_ref[...]
        acc = jnp.dot(sf_ref[...], sc, preferred_element_type=_F32)
        den = jnp.dot(sd_ref[...], sc, preferred_element_type=_F32)
        act = acc / (den + 1e-16) + b_ref[...]
        act = jnp.where(act > 0, act, jnp.exp(act) - 1.0)
        h = jnp.dot(w_ref[...], act, preferred_element_type=_F32)
        a_s = jnp.dot(ms_ref[...], h, preferred_element_type=_F32)
        a_d = jnp.dot(md_ref[...], h, preferred_element_type=_F32)
        h_ref[...] = jnp.concatenate([h, jnp.ones((1, h.shape[1]), _F32)], axis=0)
        a_ref[...] = jnp.concatenate([a_s, a_d], axis=0)

    grid = (NP // _BN,)
    full = lambda a: pl.BlockSpec(a.shape, lambda j: (0, 0))
    col = lambda r: pl.BlockSpec((r, _BN), lambda j: (0, j))
    return pl.pallas_call(
        body,
        grid=grid,
        in_specs=[col(sc_out.shape[0]), full(self_f), full(self_d), full(b_col),
                  full(Wt), full(Ms), full(Md)],
        out_specs=[col(HC + 1), col(2 * H)],
        out_shape=[
            jax.ShapeDtypeStruct((HC + 1, NP), _F32),
            jax.ShapeDtypeStruct((2 * H, NP), _F32),
        ],
    )(sc_out, self_f, self_d, b_col, Wt, Ms, Md)


def _tc_final(sc_out, self_f, self_d, b_col):
    NP = sc_out.shape[1]
    CLS = self_f.shape[0]

    def body(sc_ref, sf_ref, sd_ref, b_ref, o_ref):
        sc = sc_ref[...]
        acc = jnp.dot(sf_ref[...], sc, preferred_element_type=_F32)
        den = jnp.dot(sd_ref[...], sc, preferred_element_type=_F32)
        val = acc / (den + 1e-16) + b_ref[...]
        m = jnp.max(val, axis=0, keepdims=True)
        ls = m + jnp.log(jnp.sum(jnp.exp(val - m), axis=0, keepdims=True))
        o_ref[...] = val - ls

    grid = (NP // _BN,)
    full = lambda a: pl.BlockSpec(a.shape, lambda j: (0, 0))
    col = lambda r: pl.BlockSpec((r, _BN), lambda j: (0, j))
    return pl.pallas_call(
        body,
        grid=grid,
        in_specs=[col(sc_out.shape[0]), full(self_f), full(self_d), full(b_col)],
        out_specs=col(CLS),
        out_shape=jax.ShapeDtypeStruct((CLS, NP), _F32),
    )(sc_out, self_f, self_d, b_col)


def _sc_layer(H, C, TPH, T, KMAX, NP, EP):
    VF = H * (C + 1)
    HC = H * C
    S = H * TPH
    EPT = EP // T
    CH = 2048 if T == 1 else 1024
    NCHUNK = EPT // CH
    assert EPT % CH == 0 and NCHUNK % 2 == 0 and S * T <= _NC * _NS
    NPAIR = NCHUNK // 2
    NGRP = CH // _LN

    mesh = plsc.VectorSubcoreMesh(core_axis_name="c", subcore_axis_name="s",
                                  num_cores=_NC, num_subcores=_NS)

    @functools.partial(
        pl.kernel,
        out_type=jax.ShapeDtypeStruct((T * VF, NP), _F32),
        mesh=mesh,
        compiler_params=pltpu.CompilerParams(needs_layout_passes=False,
                                             use_tc_tiling_on_sc=False),
        scratch_types=[
            pltpu.VMEM((NP,), _F32),
            pltpu.VMEM((NP,), _F32),
            pltpu.VMEM((KMAX, NP), _F32),
            pltpu.VMEM((KMAX, NP), _F32),
            pltpu.VMEM((4, CH), jnp.int32),
            pltpu.SemaphoreType.DMA((4,)),
        ],
    )
    def k(hT, aT, srcp, dstp, out, asrc_v, adst_v, hrow_v, acc_v, ebuf, sems):
        cid = lax.axis_index("c")
        sid = lax.axis_index("s")
        wid = sid * _NC + cid
        active = wid < S * T
        t = jnp.minimum(wid // S, T - 1)
        r = wid % S
        h = jnp.minimum(r // TPH, H - 1)
        slot = r % TPH
        e_base = t * EPT

        pltpu.sync_copy(aT.at[h], asrc_v)
        pltpu.sync_copy(aT.at[H + h], adst_v)
        for kk in range(KMAX):
            vf = slot + kk * TPH
            fr = jnp.where(vf < C, h * C + vf, HC)
            pltpu.sync_copy(hT.at[fr], hrow_v.at[kk])

        def zbody(i, _):
            z = jnp.zeros((_LN,), _F32)
            for kk in range(KMAX):
                acc_v[kk, pl.ds(i * _LN, _LN)] = z
            return 0
        lax.fori_loop(0, NP // _LN, zbody, 0)

        def start(buf, c):
            off = e_base + c * CH
            pltpu.make_async_copy(srcp.at[pl.ds(off, CH)], ebuf.at[buf],
                                  sems.at[buf]).start()
            pltpu.make_async_copy(dstp.at[pl.ds(off, CH)], ebuf.at[2 + buf],
                                  sems.at[2 + buf]).start()

        def wait(buf):
            pltpu.make_async_copy(srcp.at[pl.ds(0, CH)], ebuf.at[buf],
                                  sems.at[buf]).wait()
            pltpu.make_async_copy(dstp.at[pl.ds(0, CH)], ebuf.at[2 + buf],
                                  sems.at[2 + buf]).wait()

        def process(buf):
            @plsc.parallel_loop(0, NGRP, 1, unroll=4)
            def g_body(g):
                off = g * _LN
                si = ebuf[buf, pl.ds(off, _LN)]
                di = ebuf[2 + buf, pl.ds(off, _LN)]
                av = plsc.load_gather(asrc_v, [si]) + plsc.load_gather(adst_v, [di])
                av = jnp.where(av > 0, av, 0.2 * av)
                s = jnp.exp(av)
                for kk in range(KMAX):
                    rk = jnp.full((_LN,), kk, jnp.int32)
                    gv = plsc.load_gather(hrow_v, [rk, si])
                    plsc.addupdate_scatter(acc_v, [rk, di], gv * s)

        start(0, 0)

        def pair_body(p, _):
            c0 = p * 2
            wait(0)
            start(1, c0 + 1)
            process(0)
            wait(1)

            @pl.when(p < NPAIR - 1)
            def _():
                start(0, c0 + 2)
            process(1)
            return 0
        lax.fori_loop(0, NPAIR, pair_body, 0)

        for kk in range(KMAX):
            vf = slot + kk * TPH
            row = t * VF + h * (C + 1) + jnp.minimum(vf, C)

            @pl.when(active & (vf <= C))
            def _():
                pltpu.sync_copy(acc_v.at[kk], out.at[row])

    return k


def _sel_mats(H, C, T):
    VF = H * (C + 1)
    HC = H * C
    sf = np.zeros((HC, T * VF), np.float32)
    sd = np.zeros((HC, T * VF), np.float32)
    for hh in range(H):
        for cc in range(C):
            f = hh * C + cc
            for tt in range(T):
                sf[f, tt * VF + hh * (C + 1) + cc] = 1.0
                sd[f, tt * VF + hh * (C + 1) + C] = 1.0
    return jnp.asarray(sf), jnp.asarray(sd)


def _head_sum(H, C):
    p = np.zeros((H, H * C), np.float32)
    for hh in range(H):
        p[hh, hh * C:(hh + 1) * C] = 1.0
    return jnp.asarray(p)


_CFGS = [
    (10, 10, 3, 1, 4),
    (8, 10, 4, 1, 3),
    (4, 10, 4, 2, 3),
    (1, 10, 11, 2, 1),
    (1, 10, 11, 2, 1),
]


def kernel(x, edge_index, W1, a1s, a1d, b1, W2, a2s, a2d, b2,
           W3, a3s, a3d, b3, W4, a4s, a4d, b4, W5, a5s, a5d, b5):
    N, Fin = x.shape
    E0 = edge_index.shape[1]
    NP = ((N + _BN) // _BN) * _BN
    E_tot = E0 + N
    EP = ((E_tot + 4095) // 4096) * 4096

    idx = edge_index.astype(jnp.int32)
    ar = jnp.arange(N, dtype=jnp.int32)
    pad_e = jnp.full((EP - E_tot,), N, jnp.int32)
    srcp = jnp.concatenate([idx[0], ar, pad_e])
    dstp = jnp.concatenate([idx[1], ar, pad_e])

    xT = jnp.zeros((Fin, NP), _F32).at[:, :N].set(x.T)

    Ws = [W1, W2, W3, W4, W5]
    As = [a1s, a2s, a3s, a4s, a5s]
    Ad = [a1d, a2d, a3d, a4d, a5d]
    Bs = [b1, b2, b3, b4, b5]

    sc_out = None
    for i, (H, C, TPH, T, KMAX) in enumerate(_CFGS):
        Wt = Ws[i].T
        P = _head_sum(H, C)
        Ms = P * As[i].reshape(1, -1)
        Md = P * Ad[i].reshape(1, -1)
        if i == 0:
            hT, aT = _tc_first(xT, Wt, Ms, Md)
        else:
            Hp, Cp, _, Tp, _ = _CFGS[i - 1]
            sf, sd = _sel_mats(Hp, Cp, Tp)
            b_col = Bs[i - 1].reshape(-1, 1)
            hT, aT = _tc_mid(sc_out, sf, sd, b_col, Wt, Ms, Md)
        sc_out = _sc_layer(H, C, TPH, T, KMAX, NP, EP)(hT, aT, srcp, dstp)

    H, C, _, T, _ = _CFGS[-1]
    sf, sd = _sel_mats(H, C, T)
    logitsT = _tc_final(sc_out, sf, sd, b5.reshape(-1, 1))
    return logitsT.T[:N]
